# baseline (speedup 1.0000x reference)
import functools

import jax
import jax.numpy as jnp
from jax.experimental import pallas as pl
from jax.experimental.pallas import tpu as pltpu

_TOPK = 2
_NEG_INF = float("-inf")


def _gate_kernel(x_ref, wt_ref, idx_ref, wgt_ref):
    xb = x_ref[...]
    wt = wt_ref[...]
    logits = jax.lax.dot_general(
        xb, wt, (((1,), (0,)), ((), ())),
        preferred_element_type=jnp.float32)

    blk, e = logits.shape
    lane = jax.lax.broadcasted_iota(jnp.int32, (blk, e), 1)

    m1 = jnp.max(logits, axis=-1, keepdims=True)
    i1 = jnp.min(jnp.where(logits == m1, lane, e), axis=-1,
                 keepdims=True)
    masked = jnp.where(lane == i1, _NEG_INF, logits)
    m2 = jnp.max(masked, axis=-1, keepdims=True)
    i2 = jnp.min(jnp.where(masked == m2, lane, e), axis=-1,
                 keepdims=True)

    z = jnp.sum(jnp.exp(logits - m1), axis=-1, keepdims=True)
    s1 = 1.0 / z
    s2 = jnp.exp(m2 - m1) / z
    denom = s1 + s2 + 1e-9
    w1 = s1 / denom
    w2 = s2 / denom

    idx_ref[...] = jnp.concatenate([i1, i2], axis=-1).astype(jnp.int32)
    wgt_ref[...] = jnp.concatenate([w1, w2], axis=-1)


@functools.partial(jax.jit, static_argnames=("blk",))
def _gate(x2, wt, blk):
    n, d = x2.shape
    e = wt.shape[1]
    grid = n // blk
    return pl.pallas_call(
        _gate_kernel,
        grid=(grid,),
        in_specs=[
            pl.BlockSpec((blk, d), lambda i: (i, 0)),
            pl.BlockSpec((d, e), lambda i: (0, 0)),
        ],
        out_specs=[
            pl.BlockSpec((blk, _TOPK), lambda i: (i, 0)),
            pl.BlockSpec((blk, _TOPK), lambda i: (i, 0)),
        ],
        out_shape=[
            jax.ShapeDtypeStruct((n, _TOPK), jnp.int32),
            jax.ShapeDtypeStruct((n, _TOPK), jnp.float32),
        ],
    )(x2, wt)


def kernel(x, W):
    b, s, d = x.shape
    x2 = x.reshape(b * s, d)
    idx, wgt = _gate(x2, W.T, 512)
    return idx, wgt


# speedup vs baseline: 1.2207x; 1.2207x over previous
import functools

import jax
import jax.numpy as jnp
from jax.experimental import pallas as pl
from jax.experimental.pallas import tpu as pltpu

_TOPK = 2
_NEG_INF = float("-inf")


def _gate_kernel(x_ref, wt_ref, idx_ref, wgt_ref):
    xb = x_ref[...]
    wt = wt_ref[...]
    logits = jax.lax.dot_general(
        xb, wt, (((1,), (0,)), ((), ())),
        preferred_element_type=jnp.float32)

    blk, e = logits.shape
    lane = jax.lax.broadcasted_iota(jnp.int32, (blk, e), 1)

    m1 = jnp.max(logits, axis=-1, keepdims=True)
    i1 = jnp.min(jnp.where(logits == m1, lane, e), axis=-1,
                 keepdims=True)
    masked = jnp.where(lane == i1, _NEG_INF, logits)
    m2 = jnp.max(masked, axis=-1, keepdims=True)
    i2 = jnp.min(jnp.where(masked == m2, lane, e), axis=-1,
                 keepdims=True)

    z = jnp.sum(jnp.exp(logits - m1), axis=-1, keepdims=True)
    s1 = 1.0 / z
    s2 = jnp.exp(m2 - m1) / z
    denom = s1 + s2 + 1e-9
    w1 = s1 / denom
    w2 = s2 / denom

    idx_ref[...] = jnp.concatenate([i1, i2], axis=-1).astype(jnp.int32)
    wgt_ref[...] = jnp.concatenate([w1, w2], axis=-1)


@functools.partial(jax.jit, static_argnames=("blk",))
def _gate(x2, wt, blk):
    n, d = x2.shape
    e = wt.shape[1]
    grid = n // blk
    return pl.pallas_call(
        _gate_kernel,
        grid=(grid,),
        in_specs=[
            pl.BlockSpec((blk, d), lambda i: (i, 0)),
            pl.BlockSpec((d, e), lambda i: (0, 0)),
        ],
        out_specs=[
            pl.BlockSpec((blk, _TOPK), lambda i: (i, 0)),
            pl.BlockSpec((blk, _TOPK), lambda i: (i, 0)),
        ],
        out_shape=[
            jax.ShapeDtypeStruct((n, _TOPK), jnp.int32),
            jax.ShapeDtypeStruct((n, _TOPK), jnp.float32),
        ],
    )(x2, wt)


def kernel(x, W):
    b, s, d = x.shape
    x2 = x.reshape(b * s, d)
    idx, wgt = _gate(x2, W.T, 2048)
    return idx, wgt


# speedup vs baseline: 1.2374x; 1.0136x over previous
import functools

import jax
import jax.numpy as jnp
from jax.experimental import pallas as pl
from jax.experimental.pallas import tpu as pltpu

_TOPK = 2
_NEG_INF = float("-inf")


def _gate_kernel(x_ref, wt_ref, idx_ref, wgt_ref):
    xb = x_ref[...]
    wt = wt_ref[...]
    logits = jax.lax.dot_general(
        xb, wt, (((1,), (0,)), ((), ())),
        preferred_element_type=jnp.float32)

    lt = logits.T
    e, blk = lt.shape
    row = jax.lax.broadcasted_iota(jnp.int32, (e, blk), 0)

    m1 = jnp.max(lt, axis=0, keepdims=True)
    i1 = jnp.min(jnp.where(lt == m1, row, e), axis=0, keepdims=True)
    masked = jnp.where(row == i1, _NEG_INF, lt)
    m2 = jnp.max(masked, axis=0, keepdims=True)
    i2 = jnp.min(jnp.where(masked == m2, row, e), axis=0, keepdims=True)

    z = jnp.sum(jnp.exp(lt - m1), axis=0, keepdims=True)
    s1 = 1.0 / z
    s2 = jnp.exp(m2 - m1) / z
    denom = s1 + s2 + 1e-9
    w1 = s1 / denom
    w2 = s2 / denom

    idx_ref[...] = jnp.concatenate([i1, i2], axis=0).astype(jnp.int32).T
    wgt_ref[...] = jnp.concatenate([w1, w2], axis=0).T


@functools.partial(jax.jit, static_argnames=("blk",))
def _gate(x2, wt, blk):
    n, d = x2.shape
    e = wt.shape[1]
    grid = n // blk
    return pl.pallas_call(
        _gate_kernel,
        grid=(grid,),
        in_specs=[
            pl.BlockSpec((blk, d), lambda i: (i, 0)),
            pl.BlockSpec((d, e), lambda i: (0, 0)),
        ],
        out_specs=[
            pl.BlockSpec((blk, _TOPK), lambda i: (i, 0)),
            pl.BlockSpec((blk, _TOPK), lambda i: (i, 0)),
        ],
        out_shape=[
            jax.ShapeDtypeStruct((n, _TOPK), jnp.int32),
            jax.ShapeDtypeStruct((n, _TOPK), jnp.float32),
        ],
    )(x2, wt)


def kernel(x, W):
    b, s, d = x.shape
    x2 = x.reshape(b * s, d)
    idx, wgt = _gate(x2, W.T, 2048)
    return idx, wgt


# speedup vs baseline: 1.2692x; 1.0257x over previous
import functools

import jax
import jax.numpy as jnp
from jax.experimental import pallas as pl
from jax.experimental.pallas import tpu as pltpu

_TOPK = 2
_NEG_INF = float("-inf")


def _gate_kernel(x_ref, wt_ref, idx_ref, wgt_ref):
    xb = x_ref[...]
    wt = wt_ref[...]
    logits = jax.lax.dot_general(
        xb, wt, (((1,), (0,)), ((), ())),
        preferred_element_type=jnp.float32)

    lt = logits.T
    e, blk = lt.shape
    row = jax.lax.broadcasted_iota(jnp.int32, (e, blk), 0)

    m1 = jnp.max(lt, axis=0, keepdims=True)
    i1 = jnp.min(jnp.where(lt == m1, row, e), axis=0, keepdims=True)
    masked = jnp.where(row == i1, _NEG_INF, lt)
    m2 = jnp.max(masked, axis=0, keepdims=True)
    i2 = jnp.min(jnp.where(masked == m2, row, e), axis=0, keepdims=True)

    z = jnp.sum(jnp.exp(lt - m1), axis=0, keepdims=True)
    s1 = 1.0 / z
    s2 = jnp.exp(m2 - m1) / z
    denom = s1 + s2 + 1e-9
    w1 = s1 / denom
    w2 = s2 / denom

    idx_ref[...] = jnp.concatenate([i1, i2], axis=0).astype(jnp.int32).T
    wgt_ref[...] = jnp.concatenate([w1, w2], axis=0).T


@functools.partial(jax.jit, static_argnames=("blk",))
def _gate(x2, wt, blk):
    n, d = x2.shape
    e = wt.shape[1]
    grid = n // blk
    return pl.pallas_call(
        _gate_kernel,
        grid=(grid,),
        in_specs=[
            pl.BlockSpec((blk, d), lambda i: (i, 0)),
            pl.BlockSpec((d, e), lambda i: (0, 0)),
        ],
        out_specs=[
            pl.BlockSpec((blk, _TOPK), lambda i: (i, 0)),
            pl.BlockSpec((blk, _TOPK), lambda i: (i, 0)),
        ],
        out_shape=[
            jax.ShapeDtypeStruct((n, _TOPK), jnp.int32),
            jax.ShapeDtypeStruct((n, _TOPK), jnp.float32),
        ],
    )(x2, wt)


def kernel(x, W):
    b, s, d = x.shape
    x2 = x.reshape(b * s, d)
    idx, wgt = _gate(x2, W.T, 1024)
    return idx, wgt


# speedup vs baseline: 1.3232x; 1.0426x over previous
import functools

import jax
import jax.numpy as jnp
from jax.experimental import pallas as pl
from jax.experimental.pallas import tpu as pltpu

_TOPK = 2
_NEG_INF = float("-inf")


def _gate_kernel(x_ref, w_ref, idx_ref, wgt_ref):
    xb = x_ref[...]
    w = w_ref[...]
    logits = jax.lax.dot_general(
        xb, w, (((1,), (1,)), ((), ())),
        preferred_element_type=jnp.float32)

    lt = logits.T
    e, blk = lt.shape
    row = jax.lax.broadcasted_iota(jnp.int32, (e, blk), 0)

    m1 = jnp.max(lt, axis=0, keepdims=True)
    i1 = jnp.min(jnp.where(lt == m1, row, e), axis=0, keepdims=True)
    masked = jnp.where(row == i1, _NEG_INF, lt)
    m2 = jnp.max(masked, axis=0, keepdims=True)
    i2 = jnp.min(jnp.where(masked == m2, row, e), axis=0, keepdims=True)

    z = jnp.sum(jnp.exp(lt - m1), axis=0, keepdims=True)
    s1 = 1.0 / z
    s2 = jnp.exp(m2 - m1) / z
    denom = s1 + s2 + 1e-9
    w1 = s1 / denom
    w2 = s2 / denom

    idx_ref[...] = jnp.concatenate([i1, i2], axis=0).astype(jnp.int32).T
    wgt_ref[...] = jnp.concatenate([w1, w2], axis=0).T


@functools.partial(jax.jit, static_argnames=("blk",))
def _gate(x2, w, blk):
    n, d = x2.shape
    e = w.shape[0]
    grid = n // blk
    return pl.pallas_call(
        _gate_kernel,
        grid=(grid,),
        in_specs=[
            pl.BlockSpec((blk, d), lambda i: (i, 0)),
            pl.BlockSpec((e, d), lambda i: (0, 0)),
        ],
        out_specs=[
            pl.BlockSpec((blk, _TOPK), lambda i: (i, 0)),
            pl.BlockSpec((blk, _TOPK), lambda i: (i, 0)),
        ],
        out_shape=[
            jax.ShapeDtypeStruct((n, _TOPK), jnp.int32),
            jax.ShapeDtypeStruct((n, _TOPK), jnp.float32),
        ],
    )(x2, w)


def kernel(x, W):
    b, s, d = x.shape
    x2 = x.reshape(b * s, d)
    idx, wgt = _gate(x2, W, 1024)
    return idx, wgt


# speedup vs baseline: 1.8125x; 1.3697x over previous
import functools

import jax
import jax.numpy as jnp
from jax.experimental import pallas as pl

_TOPK = 2
_NEG_INF = float("-inf")


def _gate_kernel(x_ref, w_ref, idx_ref, wgt_ref):
    xb = x_ref[...]
    w = w_ref[...]
    logits = jax.lax.dot_general(
        xb, w, (((1,), (1,)), ((), ())),
        preferred_element_type=jnp.float32)

    lt = logits.T
    e, blk = lt.shape
    row = jax.lax.broadcasted_iota(jnp.int32, (e, blk), 0)

    m1 = jnp.max(lt, axis=0, keepdims=True)
    i1 = jnp.min(jnp.where(lt == m1, row, e), axis=0, keepdims=True)
    masked = jnp.where(row == i1, _NEG_INF, lt)
    m2 = jnp.max(masked, axis=0, keepdims=True)
    i2 = jnp.min(jnp.where(masked == m2, row, e), axis=0, keepdims=True)

    z = jnp.sum(jnp.exp(lt - m1), axis=0, keepdims=True)
    s1 = 1.0 / z
    s2 = jnp.exp(m2 - m1) / z
    denom = s1 + s2 + 1e-9
    w1 = s1 / denom
    w2 = s2 / denom

    idx = jnp.concatenate([i1, i2], axis=0).astype(jnp.int32)
    wgt = jnp.concatenate([w1, w2], axis=0)
    idx_ref[...] = idx[None]
    wgt_ref[...] = wgt[None]


@functools.partial(jax.jit, static_argnames=("blk",))
def _gate(x2, w, blk):
    n, d = x2.shape
    e = w.shape[0]
    grid = n // blk
    idx, wgt = pl.pallas_call(
        _gate_kernel,
        grid=(grid,),
        in_specs=[
            pl.BlockSpec((blk, d), lambda i: (i, 0)),
            pl.BlockSpec((e, d), lambda i: (0, 0)),
        ],
        out_specs=[
            pl.BlockSpec((1, _TOPK, blk), lambda i: (i, 0, 0)),
            pl.BlockSpec((1, _TOPK, blk), lambda i: (i, 0, 0)),
        ],
        out_shape=[
            jax.ShapeDtypeStruct((grid, _TOPK, blk), jnp.int32),
            jax.ShapeDtypeStruct((grid, _TOPK, blk), jnp.float32),
        ],
    )(x2, w)
    idx = idx.transpose(0, 2, 1).reshape(n, _TOPK)
    wgt = wgt.transpose(0, 2, 1).reshape(n, _TOPK)
    return idx, wgt


def kernel(x, W):
    b, s, d = x.shape
    x2 = x.reshape(b * s, d)
    return _gate(x2, W, 1024)
